# baseline (speedup 1.0000x reference)
import functools
import jax
import jax.numpy as jnp
from jax.experimental import pallas as pl
from jax.experimental.pallas import tpu as pltpu

N_NODES = 1024
BI = 16


def _ln(x, g, b, eps=1e-5):
    mu = jnp.mean(x, axis=-1, keepdims=True)
    var = jnp.mean((x - mu) * (x - mu), axis=-1, keepdims=True)
    return (x - mu) * jax.lax.rsqrt(var + eps) * g + b


def _lrelu(x, s):
    return jnp.where(x >= 0, x, s * x)


def _decoder_body(emb_blk_ref, emb_ref, D1_ref, d1b_ref, ga_ref, ba_ref,
                  D2_ref, d2b_ref, gb_ref, bb_ref, D3_ref, d3b_ref, out_ref):
    emb_blk = emb_blk_ref[...]
    emb = emb_ref[...]
    D1 = D1_ref[...]
    ps = jnp.dot(emb_blk, D1[:32, :], preferred_element_type=jnp.float32)
    pd = jnp.dot(emb, D1[32:, :], preferred_element_type=jnp.float32)
    t1 = ps[:, None, :] + pd[None, :, :] + d1b_ref[...][None, None, :]
    t1 = t1.reshape(BI * N_NODES, 64)
    t1 = _lrelu(_ln(t1, ga_ref[...], ba_ref[...]), 0.1)
    t2 = jnp.dot(t1, D2_ref[...], preferred_element_type=jnp.float32)
    t2 = t2 + d2b_ref[...][None, :]
    t2 = _lrelu(_ln(t2, gb_ref[...], bb_ref[...]), 0.1)
    w = jnp.sum(t2 * D3_ref[...][:, 0][None, :], axis=-1) + d3b_ref[0]
    out_ref[...] = 1.0 / (1.0 + jnp.exp(-w))


def _decode(emb, D1, d1b, ga, ba, D2, d2b, gb, bb, D3, d3b):
    n = emb.shape[0]
    grid = (n // BI,)
    flat = pl.pallas_call(
        _decoder_body,
        grid=grid,
        in_specs=[
            pl.BlockSpec((BI, 32), lambda i: (i, 0)),
            pl.BlockSpec((n, 32), lambda i: (0, 0)),
            pl.BlockSpec((64, 64), lambda i: (0, 0)),
            pl.BlockSpec((64,), lambda i: (0,)),
            pl.BlockSpec((64,), lambda i: (0,)),
            pl.BlockSpec((64,), lambda i: (0,)),
            pl.BlockSpec((64, 32), lambda i: (0, 0)),
            pl.BlockSpec((32,), lambda i: (0,)),
            pl.BlockSpec((32,), lambda i: (0,)),
            pl.BlockSpec((32,), lambda i: (0,)),
            pl.BlockSpec((32, 1), lambda i: (0, 0)),
            pl.BlockSpec((1,), lambda i: (0,)),
        ],
        out_specs=pl.BlockSpec((BI * n,), lambda i: (i,)),
        out_shape=jax.ShapeDtypeStruct((n * n,), jnp.float32),
    )(emb, emb, D1, d1b, ga, ba, D2, d2b, gb, bb, D3, d3b)
    return flat.reshape(n, n)


def _gat(x, src, dst, ea, Wl, Wr, att, We, bias, heads, ch):
    N = x.shape[0]
    loop = jnp.arange(N, dtype=src.dtype)
    mean_ea = jnp.mean(ea, axis=0, keepdims=True)
    s = jnp.concatenate([src, loop])
    d = jnp.concatenate([dst, loop])
    ea2 = jnp.concatenate([ea, jnp.broadcast_to(mean_ea, (N, ea.shape[1]))], axis=0)
    xl = (x @ Wl).reshape(N, heads, ch)
    xr = (x @ Wr).reshape(N, heads, ch)
    ee = (ea2 @ We).reshape(-1, heads, ch)
    m = jax.nn.leaky_relu(xl[s] + xr[d] + ee, 0.2)
    logits = jnp.sum(m * att[None], axis=-1)
    smax = jax.ops.segment_max(logits, d, num_segments=N)
    exl = jnp.exp(logits - smax[d])
    den = jax.ops.segment_sum(exl, d, num_segments=N)
    alpha = exl / den[d]
    out = jax.ops.segment_sum(xl[s] * alpha[..., None], d, num_segments=N)
    return out.reshape(N, heads * ch) + bias


def kernel(x, edge_index, edge_attr, W1l, W1r, a1, We1, b1, W2l, W2r, a2, We2,
           b2, W3l, W3r, a3, We3, b3, g1, be1, g2, be2, g3, be3, D1, d1b, ga,
           ba, D2, d2b, gb, bb, D3, d3b):
    src, dst = edge_index[0], edge_index[1]
    h = jax.nn.elu(_ln(_gat(x, src, dst, edge_attr, W1l, W1r, a1, We1, b1, 4, 16), g1, be1))
    h = jax.nn.elu(_ln(_gat(h, src, dst, edge_attr, W2l, W2r, a2, We2, b2, 4, 16), g2, be2))
    emb = _ln(_gat(h, src, dst, edge_attr, W3l, W3r, a3, We3, b3, 1, 32), g3, be3)
    weights = _decode(emb, D1, d1b, ga, ba, D2, d2b, gb, bb, D3, d3b)
    return (weights, emb)


# speedup vs baseline: 1.2112x; 1.2112x over previous
import functools
import jax
import jax.numpy as jnp
from jax.experimental import pallas as pl
from jax.experimental.pallas import tpu as pltpu

N_NODES = 1024
BI = 128


def _ln(x, g, b, eps=1e-5):
    mu = jnp.mean(x, axis=-1, keepdims=True)
    var = jnp.mean((x - mu) * (x - mu), axis=-1, keepdims=True)
    return (x - mu) * jax.lax.rsqrt(var + eps) * g + b


def _lrelu(x, s):
    return jnp.where(x >= 0, x, s * x)


def _ln_rows(x, g, b, eps=1e-5):
    c = x.shape[0]
    mu = jnp.sum(x, axis=0, keepdims=True) * (1.0 / c)
    xc = x - mu
    var = jnp.sum(xc * xc, axis=0, keepdims=True) * (1.0 / c)
    return xc * jax.lax.rsqrt(var + eps) * g[:, None] + b[:, None]


def _decoder_body(embT_blk_ref, embT_ref, D1sT_ref, D1dT_ref, d1b_ref, ga_ref,
                  ba_ref, D2T_ref, d2b_ref, gb_ref, bb_ref, D3_ref, d3b_ref,
                  out_ref):
    embT_blk = embT_blk_ref[...]
    embT = embT_ref[...]
    psT = jnp.dot(D1sT_ref[...], embT_blk, preferred_element_type=jnp.float32)
    pdT = jnp.dot(D1dT_ref[...], embT, preferred_element_type=jnp.float32)
    pdT = pdT + d1b_ref[...][:, None]
    D2T = D2T_ref[...]
    d3row = D3_ref[...][:, 0]
    ga = ga_ref[...]
    ba = ba_ref[...]
    gb = gb_ref[...]
    bb = bb_ref[...]
    d2b = d2b_ref[...]
    for i in range(BI):
        t1 = pdT + psT[:, i:i + 1]
        x1 = _lrelu(_ln_rows(t1, ga, ba), 0.1)
        x2 = jnp.dot(D2T, x1, preferred_element_type=jnp.float32)
        x2 = x2 + d2b[:, None]
        x2 = _lrelu(_ln_rows(x2, gb, bb), 0.1)
        w = jnp.sum(x2 * d3row[:, None], axis=0) + d3b_ref[0]
        out_ref[i, :] = 1.0 / (1.0 + jnp.exp(-w))


def _decode(emb, D1, d1b, ga, ba, D2, d2b, gb, bb, D3, d3b):
    n = emb.shape[0]
    embT = emb.T
    D1sT = D1[:32, :].T
    D1dT = D1[32:, :].T
    D2T = D2.T
    grid = (n // BI,)
    return pl.pallas_call(
        _decoder_body,
        grid=grid,
        in_specs=[
            pl.BlockSpec((32, BI), lambda i: (0, i)),
            pl.BlockSpec((32, n), lambda i: (0, 0)),
            pl.BlockSpec((64, 32), lambda i: (0, 0)),
            pl.BlockSpec((64, 32), lambda i: (0, 0)),
            pl.BlockSpec((64,), lambda i: (0,)),
            pl.BlockSpec((64,), lambda i: (0,)),
            pl.BlockSpec((64,), lambda i: (0,)),
            pl.BlockSpec((32, 64), lambda i: (0, 0)),
            pl.BlockSpec((32,), lambda i: (0,)),
            pl.BlockSpec((32,), lambda i: (0,)),
            pl.BlockSpec((32,), lambda i: (0,)),
            pl.BlockSpec((32, 1), lambda i: (0, 0)),
            pl.BlockSpec((1,), lambda i: (0,)),
        ],
        out_specs=pl.BlockSpec((BI, n), lambda i: (i, 0)),
        out_shape=jax.ShapeDtypeStruct((n, n), jnp.float32),
    )(embT, embT, D1sT, D1dT, d1b, ga, ba, D2T, d2b, gb, bb, D3, d3b)


def _gat(x, src, dst, ea, Wl, Wr, att, We, bias, heads, ch):
    N = x.shape[0]
    loop = jnp.arange(N, dtype=src.dtype)
    mean_ea = jnp.mean(ea, axis=0, keepdims=True)
    s = jnp.concatenate([src, loop])
    d = jnp.concatenate([dst, loop])
    ea2 = jnp.concatenate([ea, jnp.broadcast_to(mean_ea, (N, ea.shape[1]))], axis=0)
    xl = (x @ Wl).reshape(N, heads, ch)
    xr = (x @ Wr).reshape(N, heads, ch)
    ee = (ea2 @ We).reshape(-1, heads, ch)
    m = jax.nn.leaky_relu(xl[s] + xr[d] + ee, 0.2)
    logits = jnp.sum(m * att[None], axis=-1)
    smax = jax.ops.segment_max(logits, d, num_segments=N)
    exl = jnp.exp(logits - smax[d])
    den = jax.ops.segment_sum(exl, d, num_segments=N)
    alpha = exl / den[d]
    out = jax.ops.segment_sum(xl[s] * alpha[..., None], d, num_segments=N)
    return out.reshape(N, heads * ch) + bias


def kernel(x, edge_index, edge_attr, W1l, W1r, a1, We1, b1, W2l, W2r, a2, We2,
           b2, W3l, W3r, a3, We3, b3, g1, be1, g2, be2, g3, be3, D1, d1b, ga,
           ba, D2, d2b, gb, bb, D3, d3b):
    src, dst = edge_index[0], edge_index[1]
    h = jax.nn.elu(_ln(_gat(x, src, dst, edge_attr, W1l, W1r, a1, We1, b1, 4, 16), g1, be1))
    h = jax.nn.elu(_ln(_gat(h, src, dst, edge_attr, W2l, W2r, a2, We2, b2, 4, 16), g2, be2))
    emb = _ln(_gat(h, src, dst, edge_attr, W3l, W3r, a3, We3, b3, 1, 32), g3, be3)
    weights = _decode(emb, D1, d1b, ga, ba, D2, d2b, gb, bb, D3, d3b)
    return (weights, emb)


# speedup vs baseline: 13.5579x; 11.1941x over previous
import functools
import numpy as np
import jax
import jax.numpy as jnp
from jax.experimental import pallas as pl

N = 1024
E2 = 17408
EC = 1024
BI = 128


def _lrelu(x, s):
    return jnp.where(x >= 0, x, s * x)


def _ln_lanes(x, g, b, eps=1e-5):
    mu = jnp.mean(x, axis=-1, keepdims=True)
    xc = x - mu
    var = jnp.mean(xc * xc, axis=-1, keepdims=True)
    return xc * jax.lax.rsqrt(var + eps) * g + b


def _ln_rows(x, g, b, eps=1e-5):
    c = x.shape[0]
    mu = jnp.sum(x, axis=0, keepdims=True) * (1.0 / c)
    xc = x - mu
    var = jnp.sum(xc * xc, axis=0, keepdims=True) * (1.0 / c)
    return xc * jax.lax.rsqrt(var + eps) * g[:, None] + b[:, None]


def _prep1_body(x_ref, Wl_ref, Wr_ref, We_ref, ea_ref, xl_ref, xr_ref, ee_ref):
    x = x_ref[...]
    xl_ref[...] = jnp.dot(x, Wl_ref[...], preferred_element_type=jnp.float32)
    xr_ref[...] = jnp.dot(x, Wr_ref[...], preferred_element_type=jnp.float32)
    ee_ref[...] = jnp.dot(ea_ref[...], We_ref[...], preferred_element_type=jnp.float32)


def _prep_body(num_ref, den_ref, hm_ref, b_ref, g_ref, be_ref, Wl_ref, Wr_ref,
               We_ref, ea_ref, xl_ref, xr_ref, ee_ref):
    den64 = jnp.dot(den_ref[...], hm_ref[...], preferred_element_type=jnp.float32)
    h = num_ref[...] / den64 + b_ref[...][None, :]
    h = _ln_lanes(h, g_ref[...], be_ref[...])
    h = jnp.where(h > 0, h, jnp.exp(h) - 1.0)
    xl_ref[...] = jnp.dot(h, Wl_ref[...], preferred_element_type=jnp.float32)
    xr_ref[...] = jnp.dot(h, Wr_ref[...], preferred_element_type=jnp.float32)
    ee_ref[...] = jnp.dot(ea_ref[...], We_ref[...], preferred_element_type=jnp.float32)


def _edge_body(s_ref, d_ref, xl_ref, xr_ref, ee_ref, attc_ref, hm_ref,
               hmT_ref, num_ref, den_ref):
    k = pl.program_id(0)

    @pl.when(k == 0)
    def _():
        num_ref[...] = jnp.zeros_like(num_ref)
        den_ref[...] = jnp.zeros_like(den_ref)

    s = s_ref[...]
    d = d_ref[...]
    xl = xl_ref[...]
    xr = xr_ref[...]
    rows_e = jax.lax.broadcasted_iota(jnp.int32, (EC, N), 1)
    S = (rows_e == s[:, None]).astype(jnp.float32)
    D = (rows_e == d[:, None]).astype(jnp.float32)
    cols_n = jax.lax.broadcasted_iota(jnp.int32, (N, EC), 0)
    DT = (cols_n == d[None, :]).astype(jnp.float32)
    xl_s = jnp.dot(S, xl, preferred_element_type=jnp.float32)
    xr_d = jnp.dot(D, xr, preferred_element_type=jnp.float32)
    m = _lrelu(xl_s + xr_d + ee_ref[...], 0.2)
    logits = jnp.dot(m * attc_ref[...][None, :], hm_ref[...],
                     preferred_element_type=jnp.float32)
    exl = jnp.exp(logits)
    exlC = jnp.dot(exl, hmT_ref[...], preferred_element_type=jnp.float32)
    num_ref[...] += jnp.dot(DT, xl_s * exlC, preferred_element_type=jnp.float32)
    den_ref[...] += jnp.dot(DT, exl, preferred_element_type=jnp.float32)


def _epi3_body(num_ref, den_ref, b_ref, g_ref, be_ref, emb_ref):
    h = num_ref[...] / den_ref[...] + b_ref[...][None, :]
    emb_ref[...] = _ln_lanes(h, g_ref[...], be_ref[...])


def _full(shape):
    return pl.BlockSpec(shape, lambda *_: tuple(0 for _ in shape))


def _gat_layer(prev, s, d, ea2, Wl, Wr, att, We, heads, ch):
    C = heads * ch
    attc = att.reshape(C)
    hm = jnp.asarray(np.kron(np.eye(heads, dtype=np.float32),
                             np.ones((ch, 1), np.float32)))
    hmT = hm.T
    F = prev[1].shape[1] if prev[0] == 'x' else 64
    if prev[0] == 'x':
        x = prev[1]
        xl, xr, ee = pl.pallas_call(
            _prep1_body,
            in_specs=[_full((N, F)), _full((F, C)), _full((F, C)),
                      _full((1, C)), _full((E2, 1))],
            out_specs=[_full((N, C)), _full((N, C)), _full((E2, C))],
            out_shape=[jax.ShapeDtypeStruct((N, C), jnp.float32),
                       jax.ShapeDtypeStruct((N, C), jnp.float32),
                       jax.ShapeDtypeStruct((E2, C), jnp.float32)],
        )(x, Wl, Wr, We, ea2)
    else:
        _, num, den, hmp, bias, g, be = prev
        Hp = den.shape[1]
        xl, xr, ee = pl.pallas_call(
            _prep_body,
            in_specs=[_full((N, 64)), _full((N, Hp)), _full((Hp, 64)),
                      _full((64,)), _full((64,)), _full((64,)),
                      _full((64, C)), _full((64, C)), _full((1, C)),
                      _full((E2, 1))],
            out_specs=[_full((N, C)), _full((N, C)), _full((E2, C))],
            out_shape=[jax.ShapeDtypeStruct((N, C), jnp.float32),
                       jax.ShapeDtypeStruct((N, C), jnp.float32),
                       jax.ShapeDtypeStruct((E2, C), jnp.float32)],
        )(num, den, hmp, bias, g, be, Wl, Wr, We, ea2)
    num, den = pl.pallas_call(
        _edge_body,
        grid=(E2 // EC,),
        in_specs=[
            pl.BlockSpec((EC,), lambda k: (k,)),
            pl.BlockSpec((EC,), lambda k: (k,)),
            _full((N, C)), _full((N, C)),
            pl.BlockSpec((EC, C), lambda k: (k, 0)),
            _full((C,)), _full((C, heads)), _full((heads, C)),
        ],
        out_specs=[_full((N, C)), _full((N, heads))],
        out_shape=[jax.ShapeDtypeStruct((N, C), jnp.float32),
                   jax.ShapeDtypeStruct((N, heads), jnp.float32)],
    )(s, d, xl, xr, ee, attc, hm, hmT)
    return num, den, hm


def _decoder_body(embT_blk_ref, embT_ref, D1sT_ref, D1dT_ref, d1b_ref, ga_ref,
                  ba_ref, D2T_ref, d2b_ref, gb_ref, bb_ref, D3_ref, d3b_ref,
                  out_ref):
    psT = jnp.dot(D1sT_ref[...], embT_blk_ref[...],
                  preferred_element_type=jnp.float32)
    pdT = jnp.dot(D1dT_ref[...], embT_ref[...],
                  preferred_element_type=jnp.float32)
    pdT = pdT + d1b_ref[...][:, None]
    D2T = D2T_ref[...]
    d3row = D3_ref[...][:, 0]
    ga, ba = ga_ref[...], ba_ref[...]
    gb, bb = gb_ref[...], bb_ref[...]
    d2b = d2b_ref[...]
    for i in range(BI):
        t1 = pdT + psT[:, i:i + 1]
        x1 = _lrelu(_ln_rows(t1, ga, ba), 0.1)
        x2 = jnp.dot(D2T, x1, preferred_element_type=jnp.float32)
        x2 = x2 + d2b[:, None]
        x2 = _lrelu(_ln_rows(x2, gb, bb), 0.1)
        w = jnp.sum(x2 * d3row[:, None], axis=0) + d3b_ref[0]
        out_ref[i, :] = 1.0 / (1.0 + jnp.exp(-w))


def _decode(embT, D1, d1b, ga, ba, D2, d2b, gb, bb, D3, d3b):
    D1sT = D1[:32, :].T
    D1dT = D1[32:, :].T
    D2T = D2.T
    return pl.pallas_call(
        _decoder_body,
        grid=(N // BI,),
        in_specs=[
            pl.BlockSpec((32, BI), lambda i: (0, i)),
            _full((32, N)), _full((64, 32)), _full((64, 32)),
            _full((64,)), _full((64,)), _full((64,)),
            _full((32, 64)), _full((32,)), _full((32,)), _full((32,)),
            _full((32, 1)), _full((1,)),
        ],
        out_specs=pl.BlockSpec((BI, N), lambda i: (i, 0)),
        out_shape=jax.ShapeDtypeStruct((N, N), jnp.float32),
    )(embT, embT, D1sT, D1dT, d1b, ga, ba, D2T, d2b, gb, bb, D3, d3b)


def kernel(x, edge_index, edge_attr, W1l, W1r, a1, We1, b1, W2l, W2r, a2, We2,
           b2, W3l, W3r, a3, We3, b3, g1, be1, g2, be2, g3, be3, D1, d1b, ga,
           ba, D2, d2b, gb, bb, D3, d3b):
    src, dst = edge_index[0], edge_index[1]
    loop = jnp.arange(N, dtype=src.dtype)
    s = jnp.concatenate([src, loop])
    d = jnp.concatenate([dst, loop])
    mean_ea = jnp.mean(edge_attr, axis=0, keepdims=True)
    ea2 = jnp.concatenate([edge_attr, jnp.broadcast_to(mean_ea, (N, 1))], axis=0)

    n1, dn1, hm1 = _gat_layer(('x', x), s, d, ea2, W1l, W1r, a1, We1, 4, 16)
    n2, dn2, hm2 = _gat_layer(('nd', n1, dn1, hm1.T, b1, g1, be1), s, d, ea2,
                              W2l, W2r, a2, We2, 4, 16)
    n3, dn3, _ = _gat_layer(('nd', n2, dn2, hm2.T, b2, g2, be2), s, d, ea2,
                            W3l, W3r, a3, We3, 1, 32)
    emb = pl.pallas_call(
        _epi3_body,
        in_specs=[_full((N, 32)), _full((N, 1)), _full((32,)), _full((32,)),
                  _full((32,))],
        out_specs=_full((N, 32)),
        out_shape=jax.ShapeDtypeStruct((N, 32), jnp.float32),
    )(n3, dn3, b3, g3, be3)
    weights = _decode(emb.T, D1, d1b, ga, ba, D2, d2b, gb, bb, D3, d3b)
    return (weights, emb)


# speedup vs baseline: 14.3992x; 1.0621x over previous
import functools
import numpy as np
import jax
import jax.numpy as jnp
from jax.experimental import pallas as pl
from jax.experimental.pallas import tpu as pltpu

N = 1024
E2 = 17408
EC = 1024
BI = 128


def _lrelu(x, s):
    return jnp.where(x >= 0, x, s * x)


def _ln_lanes(x, g, b, eps=1e-5):
    mu = jnp.mean(x, axis=-1, keepdims=True)
    xc = x - mu
    var = jnp.mean(xc * xc, axis=-1, keepdims=True)
    return xc * jax.lax.rsqrt(var + eps) * g + b


def _ln_rows(x, g, b, eps=1e-5):
    c = x.shape[0]
    mu = jnp.sum(x, axis=0, keepdims=True) * (1.0 / c)
    xc = x - mu
    var = jnp.sum(xc * xc, axis=0, keepdims=True) * (1.0 / c)
    return xc * jax.lax.rsqrt(var + eps) * g[:, None] + b[:, None]


def _layer_body(first, s_ref, d_ref, ea_ref, Wl_ref, Wr_ref, We_ref,
                attc_ref, hm_ref, hmT_ref, *rest):
    if first:
        x_ref = rest[0]
        num_ref, den_ref = rest[1], rest[2]
        xl_sc, xr_sc = rest[3], rest[4]
    else:
        (pnum_ref, pden_ref, phmT_ref, pb_ref, pg_ref, pbe_ref,
         num_ref, den_ref, xl_sc, xr_sc) = rest
    k = pl.program_id(0)

    @pl.when(k == 0)
    def _():
        if first:
            h = x_ref[...]
        else:
            den64 = jnp.dot(pden_ref[...], phmT_ref[...],
                            preferred_element_type=jnp.float32)
            h = pnum_ref[...] / den64 + pb_ref[...][None, :]
            h = _ln_lanes(h, pg_ref[...], pbe_ref[...])
            h = jnp.where(h > 0, h, jnp.exp(h) - 1.0)
        xl_sc[...] = jnp.dot(h, Wl_ref[...], preferred_element_type=jnp.float32)
        xr_sc[...] = jnp.dot(h, Wr_ref[...], preferred_element_type=jnp.float32)
        num_ref[...] = jnp.zeros_like(num_ref)
        den_ref[...] = jnp.zeros_like(den_ref)

    s = s_ref[...]
    d = d_ref[...]
    ee = jnp.dot(ea_ref[...], We_ref[...], preferred_element_type=jnp.float32)
    rows_e = jax.lax.broadcasted_iota(jnp.int32, (EC, N), 1)
    S = (rows_e == s[:, None]).astype(jnp.float32)
    D = (rows_e == d[:, None]).astype(jnp.float32)
    cols_n = jax.lax.broadcasted_iota(jnp.int32, (N, EC), 0)
    DT = (cols_n == d[None, :]).astype(jnp.float32)
    xl_s = jnp.dot(S, xl_sc[...], preferred_element_type=jnp.float32)
    xr_d = jnp.dot(D, xr_sc[...], preferred_element_type=jnp.float32)
    m = _lrelu(xl_s + xr_d + ee, 0.2)
    logits = jnp.dot(m * attc_ref[...][None, :], hm_ref[...],
                     preferred_element_type=jnp.float32)
    exl = jnp.exp(logits)
    exlC = jnp.dot(exl, hmT_ref[...], preferred_element_type=jnp.float32)
    num_ref[...] += jnp.dot(DT, xl_s * exlC, preferred_element_type=jnp.float32)
    den_ref[...] += jnp.dot(DT, exl, preferred_element_type=jnp.float32)


def _full(shape):
    return pl.BlockSpec(shape, lambda *_: tuple(0 for _ in shape))


def _gat_layer(prev, s, d, ea2, Wl, Wr, att, We, heads, ch):
    C = heads * ch
    attc = att.reshape(C)
    hm = jnp.asarray(np.kron(np.eye(heads, dtype=np.float32),
                             np.ones((ch, 1), np.float32)))
    hmT = hm.T
    first = prev[0] == 'x'
    F = prev[1].shape[1] if first else 64
    common_in = [
        pl.BlockSpec((EC,), lambda k: (k,)),
        pl.BlockSpec((EC,), lambda k: (k,)),
        pl.BlockSpec((EC, 1), lambda k: (k, 0)),
        _full((F, C)), _full((F, C)), _full((1, C)),
        _full((C,)), _full((C, heads)), _full((heads, C)),
    ]
    if first:
        extra_in = [_full((N, F))]
        args = (s, d, ea2, Wl, Wr, We, attc, hm, hmT, prev[1])
    else:
        _, num_p, den_p, hmpT, bias, g, be = prev
        Hp = den_p.shape[1]
        extra_in = [_full((N, 64)), _full((N, Hp)), _full((Hp, 64)),
                    _full((64,)), _full((64,)), _full((64,))]
        args = (s, d, ea2, Wl, Wr, We, attc, hm, hmT,
                num_p, den_p, hmpT, bias, g, be)
    num, den = pl.pallas_call(
        functools.partial(_layer_body, first),
        grid=(E2 // EC,),
        in_specs=common_in + extra_in,
        out_specs=[_full((N, C)), _full((N, heads))],
        out_shape=[jax.ShapeDtypeStruct((N, C), jnp.float32),
                   jax.ShapeDtypeStruct((N, heads), jnp.float32)],
        scratch_shapes=[pltpu.VMEM((N, C), jnp.float32),
                        pltpu.VMEM((N, C), jnp.float32)],
    )(*args)
    return num, den, hmT


def _decoder_body(num3_ref, den3_ref, b3_ref, g3_ref, be3_ref, D1sT_ref,
                  D1dT_ref, d1b_ref, ga_ref, ba_ref, D2T_ref, d2b_ref,
                  gb_ref, bb_ref, D3_ref, d3b_ref, out_ref, emb_ref,
                  embT_sc):
    i_blk = pl.program_id(0)

    @pl.when(i_blk == 0)
    def _():
        h = num3_ref[...] / den3_ref[...] + b3_ref[...][None, :]
        emb = _ln_lanes(h, g3_ref[...], be3_ref[...])
        emb_ref[...] = emb
        embT_sc[...] = emb.T

    embT = embT_sc[...]
    embT_blk = embT_sc[:, pl.ds(i_blk * BI, BI)]
    psT = jnp.dot(D1sT_ref[...], embT_blk, preferred_element_type=jnp.float32)
    pdT = jnp.dot(D1dT_ref[...], embT, preferred_element_type=jnp.float32)
    pdT = pdT + d1b_ref[...][:, None]
    D2T = D2T_ref[...]
    d3row = D3_ref[...][:, 0]
    ga, ba = ga_ref[...], ba_ref[...]
    gb, bb = gb_ref[...], bb_ref[...]
    d2b = d2b_ref[...]
    for i in range(BI):
        t1 = pdT + psT[:, i:i + 1]
        x1 = _lrelu(_ln_rows(t1, ga, ba), 0.1)
        x2 = jnp.dot(D2T, x1, preferred_element_type=jnp.float32)
        x2 = x2 + d2b[:, None]
        x2 = _lrelu(_ln_rows(x2, gb, bb), 0.1)
        w = jnp.sum(x2 * d3row[:, None], axis=0) + d3b_ref[0]
        out_ref[i, :] = 1.0 / (1.0 + jnp.exp(-w))


def _decode(num3, den3, b3, g3, be3, D1, d1b, ga, ba, D2, d2b, gb, bb, D3, d3b):
    D1sT = D1[:32, :].T
    D1dT = D1[32:, :].T
    D2T = D2.T
    return pl.pallas_call(
        _decoder_body,
        grid=(N // BI,),
        in_specs=[
            _full((N, 32)), _full((N, 1)), _full((32,)), _full((32,)),
            _full((32,)),
            _full((64, 32)), _full((64, 32)),
            _full((64,)), _full((64,)), _full((64,)),
            _full((32, 64)), _full((32,)), _full((32,)), _full((32,)),
            _full((32, 1)), _full((1,)),
        ],
        out_specs=[pl.BlockSpec((BI, N), lambda i: (i, 0)), _full((N, 32))],
        out_shape=[jax.ShapeDtypeStruct((N, N), jnp.float32),
                   jax.ShapeDtypeStruct((N, 32), jnp.float32)],
        scratch_shapes=[pltpu.VMEM((32, N), jnp.float32)],
    )(num3, den3, b3, g3, be3, D1sT, D1dT, d1b, ga, ba, D2T, d2b, gb, bb,
      D3, d3b)


def kernel(x, edge_index, edge_attr, W1l, W1r, a1, We1, b1, W2l, W2r, a2, We2,
           b2, W3l, W3r, a3, We3, b3, g1, be1, g2, be2, g3, be3, D1, d1b, ga,
           ba, D2, d2b, gb, bb, D3, d3b):
    src, dst = edge_index[0], edge_index[1]
    loop = jnp.arange(N, dtype=src.dtype)
    s = jnp.concatenate([src, loop])
    d = jnp.concatenate([dst, loop])
    mean_ea = jnp.mean(edge_attr, axis=0, keepdims=True)
    ea2 = jnp.concatenate([edge_attr, jnp.broadcast_to(mean_ea, (N, 1))], axis=0)

    n1, dn1, hmT1 = _gat_layer(('x', x), s, d, ea2, W1l, W1r, a1, We1, 4, 16)
    n2, dn2, hmT2 = _gat_layer(('nd', n1, dn1, hmT1, b1, g1, be1), s, d, ea2,
                               W2l, W2r, a2, We2, 4, 16)
    n3, dn3, _ = _gat_layer(('nd', n2, dn2, hmT2, b2, g2, be2), s, d, ea2,
                            W3l, W3r, a3, We3, 1, 32)
    weights, emb = _decode(n3, dn3, b3, g3, be3, D1, d1b, ga, ba, D2, d2b,
                           gb, bb, D3, d3b)
    return (weights, emb)
